# baseline (speedup 1.0000x reference)
import math

import jax
import jax.numpy as jnp
from jax.experimental import pallas as pl

DIM = 256
HEADS = 8
HEAD_DIM = DIM // HEADS
SCALE = HEAD_DIM ** (-0.5)
THRESH = 15.0
N_TOK = 2048
ROW_BLK = 2048
N_RB = N_TOK // ROW_BLK
LOG2E = math.log2(math.e)

_NAN = float("nan")


def _prep_body(x_ref, w_ref, qkvT_ref, v01_ref, nan_ref):
    x = x_ref[...]
    w = w_ref[...]
    qkvT = jax.lax.dot_general(
        w.astype(jnp.bfloat16), x.astype(jnp.bfloat16),
        (((1,), (1,)), ((), ())),
        preferred_element_type=jnp.float32)
    nrm = jnp.sqrt(jnp.sum(x * x, axis=-1, keepdims=True))
    mx = jnp.max(nrm)
    valid = (nrm / mx * 75.0) >= THRESH
    nan_ref[...] = jnp.where(valid, 0.0, _NAN).astype(jnp.float32)
    v01row = jnp.where(valid, 1.0, 0.0).reshape(1, N_TOK)
    v01_ref[...] = v01row.astype(jnp.bfloat16)
    qkvT_ref[:DIM, :] = (qkvT[:DIM, :] * (SCALE * LOG2E)).astype(jnp.bfloat16)
    qkvT_ref[DIM:2 * DIM, :] = qkvT[DIM:2 * DIM, :].astype(jnp.bfloat16)
    qkvT_ref[2 * DIM:, :] = (qkvT[2 * DIM:, :] * v01row).astype(jnp.bfloat16)


def _attn_body(q_ref, k_ref, v_ref, v01_ref, o_ref):
    q = q_ref[...]
    k = k_ref[...]
    qf = q.astype(jnp.float32)
    kf = k.astype(jnp.float32)
    qn = jnp.sqrt(jnp.sum(qf * qf, axis=0, keepdims=True))
    kn2 = jnp.sum(kf * kf, axis=0, keepdims=True)
    kmax = jnp.sqrt(jnp.max(kn2))
    m = qn * kmax
    zeros7 = jnp.zeros((7, N_TOK), jnp.bfloat16)
    q_aug = jnp.concatenate(
        [q, (-m).astype(jnp.bfloat16), zeros7[:, :ROW_BLK]], axis=0)
    k_aug = jnp.concatenate(
        [k, jnp.ones((1, N_TOK), jnp.bfloat16), zeros7], axis=0)
    t = jax.lax.dot_general(
        q_aug, k_aug, (((0,), (0,)), ((), ())),
        preferred_element_type=jnp.float32)
    e = jnp.exp2(t.astype(jnp.bfloat16))
    vcat = jnp.concatenate(
        [v_ref[...], jnp.broadcast_to(v01_ref[...], (8, N_TOK))], axis=0)
    oa = jax.lax.dot_general(
        vcat, e, (((1,), (1,)), ((), ())),
        preferred_element_type=jnp.float32)
    o = oa[:HEAD_DIM, :] / oa[HEAD_DIM:HEAD_DIM + 1, :]
    o_ref[...] = o.astype(jnp.bfloat16)


def _fin_body(outT_ref, wp_ref, b_ref, x_ref, nan_ref, o_ref):
    res = jax.lax.dot_general(
        outT_ref[...], wp_ref[...].astype(jnp.bfloat16),
        (((0,), (1,)), ((), ())),
        preferred_element_type=jnp.float32)
    o_ref[...] = res + b_ref[...] + x_ref[...] + nan_ref[...]


@jax.jit
def kernel(x, W_qkv, W_proj, b_proj):
    B, T, H, W, C = x.shape
    N = T * H * W
    x_seq = x.reshape(N, C)

    qkvT, v01, nanv = pl.pallas_call(
        _prep_body,
        out_shape=(
            jax.ShapeDtypeStruct((3 * C, N), jnp.bfloat16),
            jax.ShapeDtypeStruct((1, N), jnp.bfloat16),
            jax.ShapeDtypeStruct((N, 1), jnp.float32),
        ),
    )(x_seq, W_qkv)

    nhb = HEAD_DIM
    outT = pl.pallas_call(
        _attn_body,
        grid=(HEADS, N_RB),
        in_specs=[
            pl.BlockSpec((nhb, ROW_BLK), lambda h, rb: (h, rb)),
            pl.BlockSpec((nhb, N), lambda h, rb: (HEADS + h, 0)),
            pl.BlockSpec((nhb, N), lambda h, rb: (2 * HEADS + h, 0)),
            pl.BlockSpec((1, N), lambda h, rb: (0, 0)),
        ],
        out_specs=pl.BlockSpec((nhb, ROW_BLK), lambda h, rb: (h, rb)),
        out_shape=jax.ShapeDtypeStruct((C, N), jnp.bfloat16),
    )(qkvT, qkvT, qkvT, v01)

    out = pl.pallas_call(
        _fin_body,
        out_shape=jax.ShapeDtypeStruct((N, C), jnp.float32),
    )(outT, W_proj, b_proj.reshape(1, C), x_seq, nanv)

    return out.reshape(B, T, H, W, C)


# speedup vs baseline: 1.1268x; 1.1268x over previous
import math

import jax
import jax.numpy as jnp
from jax.experimental import pallas as pl
from jax.experimental.pallas import tpu as pltpu

DIM = 256
HEADS = 8
HEAD_DIM = DIM // HEADS
SCALE = HEAD_DIM ** (-0.5)
THRESH = 15.0
N_TOK = 2048
LOG2E = math.log2(math.e)

_NAN = float("nan")


def _body(x_ref, wqkv_ref, wp_ref, b_ref, o_ref,
          qkvT_ref, v01_ref, nan_ref, outT_ref):
    i = pl.program_id(0)

    @pl.when(i == 0)
    def _prep():
        x = x_ref[...]
        w = wqkv_ref[...]
        qkvT = jax.lax.dot_general(
            w.astype(jnp.bfloat16), x.astype(jnp.bfloat16),
            (((1,), (1,)), ((), ())),
            preferred_element_type=jnp.float32)
        nrm = jnp.sqrt(jnp.sum(x * x, axis=-1, keepdims=True))
        mx = jnp.max(nrm)
        valid = (nrm / mx * 75.0) >= THRESH
        nan_ref[...] = jnp.where(valid, 0.0, _NAN).astype(jnp.float32)
        v01row = jnp.where(valid, 1.0, 0.0).reshape(1, N_TOK)
        v01_ref[...] = v01row.astype(jnp.bfloat16)
        q = qkvT[:DIM, :] * (SCALE * LOG2E)
        k = qkvT[DIM:2 * DIM, :]
        v = qkvT[2 * DIM:, :] * v01row
        qkv = jnp.concatenate([q, k, v], axis=0).astype(jnp.bfloat16)
        qkvT_ref[...] = qkv.reshape(3 * HEADS, HEAD_DIM, N_TOK)

    @pl.when(jnp.logical_and(i >= 1, i <= HEADS))
    def _attn():
        h = i - 1
        q = qkvT_ref[h]
        k = qkvT_ref[HEADS + h]
        v = qkvT_ref[2 * HEADS + h]
        qf = q.astype(jnp.float32)
        kf = k.astype(jnp.float32)
        qn = jnp.sqrt(jnp.sum(qf * qf, axis=0, keepdims=True))
        kn2 = jnp.sum(kf * kf, axis=0, keepdims=True)
        kmax = jnp.sqrt(jnp.max(kn2))
        m = qn * kmax
        zeros7 = jnp.zeros((7, N_TOK), jnp.bfloat16)
        q_aug = jnp.concatenate(
            [q, (-m).astype(jnp.bfloat16), zeros7], axis=0)
        k_aug = jnp.concatenate(
            [k, jnp.ones((1, N_TOK), jnp.bfloat16), zeros7], axis=0)
        t = jax.lax.dot_general(
            q_aug, k_aug, (((0,), (0,)), ((), ())),
            preferred_element_type=jnp.float32)
        e = jnp.exp2(t.astype(jnp.bfloat16))
        vcat = jnp.concatenate(
            [v, jnp.broadcast_to(v01_ref[...], (8, N_TOK))], axis=0)
        oa = jax.lax.dot_general(
            vcat, e, (((1,), (1,)), ((), ())),
            preferred_element_type=jnp.float32)
        o = oa[:HEAD_DIM, :] / oa[HEAD_DIM:HEAD_DIM + 1, :]
        outT_ref[h] = o.astype(jnp.bfloat16)

    @pl.when(i == HEADS + 1)
    def _fin():
        outT = outT_ref[...].reshape(DIM, N_TOK)
        res = jax.lax.dot_general(
            outT, wp_ref[...].astype(jnp.bfloat16),
            (((0,), (1,)), ((), ())),
            preferred_element_type=jnp.float32)
        o_ref[...] = res + b_ref[...] + x_ref[...] + nan_ref[...]


@jax.jit
def kernel(x, W_qkv, W_proj, b_proj):
    B, T, H, W, C = x.shape
    N = T * H * W
    x_seq = x.reshape(N, C)

    out = pl.pallas_call(
        _body,
        grid=(HEADS + 2,),
        in_specs=[
            pl.BlockSpec((N, C), lambda i: (0, 0)),
            pl.BlockSpec((3 * C, C), lambda i: (0, 0)),
            pl.BlockSpec((C, C), lambda i: (0, 0)),
            pl.BlockSpec((1, C), lambda i: (0, 0)),
        ],
        out_specs=pl.BlockSpec((N, C), lambda i: (0, 0)),
        out_shape=jax.ShapeDtypeStruct((N, C), jnp.float32),
        scratch_shapes=[
            pltpu.VMEM((3 * HEADS, HEAD_DIM, N_TOK), jnp.bfloat16),
            pltpu.VMEM((1, N_TOK), jnp.bfloat16),
            pltpu.VMEM((N_TOK, 1), jnp.float32),
            pltpu.VMEM((HEADS, HEAD_DIM, N_TOK), jnp.bfloat16),
        ],
    )(x_seq, W_qkv, W_proj, b_proj.reshape(1, C))

    return out.reshape(B, T, H, W, C)


# speedup vs baseline: 1.1481x; 1.0189x over previous
import math

import jax
import jax.numpy as jnp
from jax.experimental import pallas as pl
from jax.experimental.pallas import tpu as pltpu

DIM = 256
HEADS = 8
HEAD_DIM = DIM // HEADS
SCALE = HEAD_DIM ** (-0.5)
THRESH = 15.0
N_TOK = 2048
LOG2E = math.log2(math.e)

_NAN = float("nan")


def _body(x_ref, wqkv_ref, wp_ref, b_ref, o_ref,
          qkvT_ref, v01_ref, nan_ref, outT_ref):
    i = pl.program_id(0)

    @pl.when(i == 0)
    def _prep():
        x = x_ref[...]
        w = wqkv_ref[...]
        qkvT = jax.lax.dot_general(
            w.astype(jnp.bfloat16), x.astype(jnp.bfloat16),
            (((1,), (1,)), ((), ())),
            preferred_element_type=jnp.float32)
        nrm = jnp.sqrt(jnp.sum(x * x, axis=-1, keepdims=True))
        mx = jnp.max(nrm)
        valid = (nrm / mx * 75.0) >= THRESH
        nan_ref[...] = jnp.where(valid, 0.0, _NAN).astype(jnp.float32)
        v01row = jnp.where(valid, 1.0, 0.0).reshape(1, N_TOK)
        v01_ref[...] = v01row.astype(jnp.bfloat16)
        q = qkvT[:DIM, :] * (SCALE * LOG2E)
        k = qkvT[DIM:2 * DIM, :]
        v = qkvT[2 * DIM:, :] * v01row
        qkv = jnp.concatenate([q, k, v], axis=0).astype(jnp.bfloat16)
        qkvT_ref[...] = qkv.reshape(3 * HEADS, HEAD_DIM, N_TOK)

    @pl.when(jnp.logical_and(i >= 1, i <= HEADS))
    def _attn():
        h = i - 1
        q = qkvT_ref[h]
        k = qkvT_ref[HEADS + h]
        v = qkvT_ref[2 * HEADS + h]
        qf = q.astype(jnp.float32)
        kf = k.astype(jnp.float32)
        qn = jnp.sqrt(jnp.sum(qf * qf, axis=0, keepdims=True))
        kn2 = jnp.sum(kf * kf, axis=0, keepdims=True)
        kmax = jnp.sqrt(jnp.max(kn2))
        m = qn * kmax
        zeros7 = jnp.zeros((7, N_TOK), jnp.bfloat16)
        q_aug = jnp.concatenate(
            [q, (-m).astype(jnp.bfloat16), zeros7], axis=0)
        k_aug = jnp.concatenate(
            [k, jnp.ones((1, N_TOK), jnp.bfloat16), zeros7], axis=0)
        vcat = jnp.concatenate(
            [v, jnp.broadcast_to(v01_ref[...], (8, N_TOK))], axis=0)
        CB = N_TOK // 4
        oa = jnp.zeros((HEAD_DIM + 8, N_TOK), jnp.float32)
        for cb in range(4):
            ksl = k_aug[:, cb * CB:(cb + 1) * CB]
            t = jax.lax.dot_general(
                q_aug, ksl, (((0,), (0,)), ((), ())),
                preferred_element_type=jnp.float32)
            e = jnp.exp2(t.astype(jnp.bfloat16))
            vsl = vcat[:, cb * CB:(cb + 1) * CB]
            oa = oa + jax.lax.dot_general(
                vsl, e, (((1,), (1,)), ((), ())),
                preferred_element_type=jnp.float32)
        o = oa[:HEAD_DIM, :] / oa[HEAD_DIM:HEAD_DIM + 1, :]
        outT_ref[h] = o.astype(jnp.bfloat16)

    @pl.when(i == HEADS + 1)
    def _fin():
        outT = outT_ref[...].reshape(DIM, N_TOK)
        res = jax.lax.dot_general(
            outT, wp_ref[...].astype(jnp.bfloat16),
            (((0,), (1,)), ((), ())),
            preferred_element_type=jnp.float32)
        o_ref[...] = res + b_ref[...] + x_ref[...] + nan_ref[...]


@jax.jit
def kernel(x, W_qkv, W_proj, b_proj):
    B, T, H, W, C = x.shape
    N = T * H * W
    x_seq = x.reshape(N, C)

    out = pl.pallas_call(
        _body,
        grid=(HEADS + 2,),
        in_specs=[
            pl.BlockSpec((N, C), lambda i: (0, 0)),
            pl.BlockSpec((3 * C, C), lambda i: (0, 0)),
            pl.BlockSpec((C, C), lambda i: (0, 0)),
            pl.BlockSpec((1, C), lambda i: (0, 0)),
        ],
        out_specs=pl.BlockSpec((N, C), lambda i: (0, 0)),
        out_shape=jax.ShapeDtypeStruct((N, C), jnp.float32),
        scratch_shapes=[
            pltpu.VMEM((3 * HEADS, HEAD_DIM, N_TOK), jnp.bfloat16),
            pltpu.VMEM((1, N_TOK), jnp.bfloat16),
            pltpu.VMEM((N_TOK, 1), jnp.float32),
            pltpu.VMEM((HEADS, HEAD_DIM, N_TOK), jnp.bfloat16),
        ],
    )(x_seq, W_qkv, W_proj, b_proj.reshape(1, C))

    return out.reshape(B, T, H, W, C)
